# Initial kernel scaffold; baseline (speedup 1.0000x reference)
import jax
import jax.numpy as jnp
from jax.experimental import pallas as pl


def kernel(x, table, W1, b1, W2, b2):
    raise NotImplementedError("write your pallas kernel here")


# speedup vs baseline: 1.5681x; 1.5681x over previous
import jax
import jax.numpy as jnp
from jax.experimental import pallas as pl

_D_MODEL = 768
_D_HID = 384
_N_EXP = 64
_BLK = 1024


def _router_body(x_ref, emb_ref, w1_ref, b1_ref, w2_ref, b2_ref,
                 w_out_ref, idx_out_ref):
    xc = x_ref[...] + emb_ref[...]
    h = jnp.dot(xc, w1_ref[...], preferred_element_type=jnp.float32)
    h = jnp.maximum(h + b1_ref[...], 0.0)
    logits = jnp.dot(h, w2_ref[...], preferred_element_type=jnp.float32)
    logits = logits + b2_ref[...]

    m = jnp.max(logits, axis=-1, keepdims=True)
    e = jnp.exp(logits - m)
    w = e / jnp.sum(e, axis=-1, keepdims=True)
    w_out_ref[...] = w

    idx = jax.lax.broadcasted_iota(jnp.int32, w.shape, 1)
    big = jnp.int32(_N_EXP)
    m1 = jnp.max(w, axis=-1, keepdims=True)
    i1 = jnp.min(jnp.where(w == m1, idx, big), axis=-1, keepdims=True)
    w_masked = jnp.where(idx == i1, -jnp.inf, w)
    m2 = jnp.max(w_masked, axis=-1, keepdims=True)
    i2 = jnp.min(jnp.where(w_masked == m2, idx, big), axis=-1, keepdims=True)
    idx_out_ref[...] = jnp.concatenate([i1, i2], axis=-1)


def kernel(x, table, W1, b1, W2, b2):
    batch, seq, d_model = x.shape
    n_tok = batch * seq
    x2 = x.reshape(n_tok, d_model)
    emb = table[0].reshape(1, d_model)
    b1r = b1.reshape(1, _D_HID)
    b2r = b2.reshape(1, _N_EXP)

    grid = (n_tok // _BLK,)
    weights, indices = pl.pallas_call(
        _router_body,
        grid=grid,
        in_specs=[
            pl.BlockSpec((_BLK, d_model), lambda i: (i, 0)),
            pl.BlockSpec((1, d_model), lambda i: (0, 0)),
            pl.BlockSpec((d_model, _D_HID), lambda i: (0, 0)),
            pl.BlockSpec((1, _D_HID), lambda i: (0, 0)),
            pl.BlockSpec((_D_HID, _N_EXP), lambda i: (0, 0)),
            pl.BlockSpec((1, _N_EXP), lambda i: (0, 0)),
        ],
        out_specs=[
            pl.BlockSpec((_BLK, _N_EXP), lambda i: (i, 0)),
            pl.BlockSpec((_BLK, 2), lambda i: (i, 0)),
        ],
        out_shape=[
            jax.ShapeDtypeStruct((n_tok, _N_EXP), jnp.float32),
            jax.ShapeDtypeStruct((n_tok, 2), jnp.int32),
        ],
    )(x2, emb, W1, b1r, W2, b2r)

    return (weights.reshape(batch, seq, _N_EXP),
            indices.reshape(batch, seq, 2))


# speedup vs baseline: 1.6494x; 1.0519x over previous
import jax
import jax.numpy as jnp
from jax.experimental import pallas as pl

_D_MODEL = 768
_D_HID = 384
_N_EXP = 64
_BLK = 1024


def _router_body(x_ref, w1_ref, b1_ref, w2_ref, b2_ref,
                 w_out_ref, idx_out_ref):
    h = jnp.dot(x_ref[...], w1_ref[...], preferred_element_type=jnp.float32)
    h = jnp.maximum(h + b1_ref[...], 0.0)
    logits = jnp.dot(h, w2_ref[...], preferred_element_type=jnp.float32)
    logits = logits + b2_ref[...]

    m = jnp.max(logits, axis=-1, keepdims=True)
    e = jnp.exp(logits - m)
    s = jnp.sum(e, axis=-1, keepdims=True)
    w_out_ref[...] = e / s

    idx = jax.lax.broadcasted_iota(jnp.int32, e.shape, 1)
    enc = (jax.lax.bitcast_convert_type(e, jnp.int32)
           & jnp.int32(~63)) | (jnp.int32(63) - idx)
    e1 = jnp.max(enc, axis=-1, keepdims=True)
    i1 = jnp.int32(63) - (e1 & jnp.int32(63))
    enc2 = jnp.where(idx == i1, jnp.int32(-1), enc)
    e2 = jnp.max(enc2, axis=-1, keepdims=True)
    i2 = jnp.int32(63) - (e2 & jnp.int32(63))
    idx_out_ref[...] = jnp.concatenate([i1, i2], axis=-1)


def kernel(x, table, W1, b1, W2, b2):
    batch, seq, d_model = x.shape
    n_tok = batch * seq
    x2 = x.reshape(n_tok, d_model)
    emb = table[0].reshape(1, d_model)
    b1_eff = b1.reshape(1, _D_HID) + emb @ W1
    b2r = b2.reshape(1, _N_EXP)

    grid = (n_tok // _BLK,)
    weights, indices = pl.pallas_call(
        _router_body,
        grid=grid,
        in_specs=[
            pl.BlockSpec((_BLK, d_model), lambda i: (i, 0)),
            pl.BlockSpec((d_model, _D_HID), lambda i: (0, 0)),
            pl.BlockSpec((1, _D_HID), lambda i: (0, 0)),
            pl.BlockSpec((_D_HID, _N_EXP), lambda i: (0, 0)),
            pl.BlockSpec((1, _N_EXP), lambda i: (0, 0)),
        ],
        out_specs=[
            pl.BlockSpec((_BLK, _N_EXP), lambda i: (i, 0)),
            pl.BlockSpec((_BLK, 2), lambda i: (i, 0)),
        ],
        out_shape=[
            jax.ShapeDtypeStruct((n_tok, _N_EXP), jnp.float32),
            jax.ShapeDtypeStruct((n_tok, 2), jnp.int32),
        ],
    )(x2, W1, b1_eff, W2, b2r)

    return (weights.reshape(batch, seq, _N_EXP),
            indices.reshape(batch, seq, 2))


# speedup vs baseline: 1.7419x; 1.0560x over previous
import jax
import jax.numpy as jnp
from jax.experimental import pallas as pl

_D_MODEL = 768
_D_HID = 384
_N_EXP = 64
_BLK = 1024


def _router_body(x_ref, w1_ref, b1_ref, w2_ref, b2_ref, ones_ref,
                 w_out_ref, idx_out_ref):
    h = jnp.dot(x_ref[...], w1_ref[...], preferred_element_type=jnp.float32)
    h = jnp.maximum(h + b1_ref[...], 0.0)
    logits = jnp.dot(h, w2_ref[...], preferred_element_type=jnp.float32)
    logits = logits + b2_ref[...]

    m = jnp.max(logits, axis=-1, keepdims=True)
    e = jnp.exp(logits - m)
    s = jnp.dot(e, ones_ref[...], preferred_element_type=jnp.float32)
    w_out_ref[...] = e / s

    idx = jax.lax.broadcasted_iota(jnp.int32, e.shape, 1)
    enc = (jax.lax.bitcast_convert_type(e, jnp.int32)
           & jnp.int32(~63)) | (jnp.int32(63) - idx)
    encf = jax.lax.bitcast_convert_type(enc, jnp.float32)
    m1 = jnp.max(encf, axis=-1, keepdims=True)
    i1 = (jnp.int32(63)
          - (jax.lax.bitcast_convert_type(m1, jnp.int32) & jnp.int32(63)))
    encf2 = jnp.where(idx == i1, jnp.float32(0.0), encf)
    m2 = jnp.max(encf2, axis=-1, keepdims=True)
    i2 = (jnp.int32(63)
          - (jax.lax.bitcast_convert_type(m2, jnp.int32) & jnp.int32(63)))
    idx_out_ref[...] = jnp.concatenate([i1, i2], axis=-1)


def kernel(x, table, W1, b1, W2, b2):
    batch, seq, d_model = x.shape
    n_tok = batch * seq
    x2 = x.reshape(n_tok, d_model)
    emb = table[0].reshape(1, d_model)
    b1_eff = b1.reshape(1, _D_HID) + emb @ W1
    b2r = b2.reshape(1, _N_EXP)
    ones = jnp.ones((_N_EXP, _N_EXP), jnp.float32)

    grid = (n_tok // _BLK,)
    weights, indices = pl.pallas_call(
        _router_body,
        grid=grid,
        in_specs=[
            pl.BlockSpec((_BLK, d_model), lambda i: (i, 0)),
            pl.BlockSpec((d_model, _D_HID), lambda i: (0, 0)),
            pl.BlockSpec((1, _D_HID), lambda i: (0, 0)),
            pl.BlockSpec((_D_HID, _N_EXP), lambda i: (0, 0)),
            pl.BlockSpec((1, _N_EXP), lambda i: (0, 0)),
            pl.BlockSpec((_N_EXP, _N_EXP), lambda i: (0, 0)),
        ],
        out_specs=[
            pl.BlockSpec((_BLK, _N_EXP), lambda i: (i, 0)),
            pl.BlockSpec((_BLK, 2), lambda i: (i, 0)),
        ],
        out_shape=[
            jax.ShapeDtypeStruct((n_tok, _N_EXP), jnp.float32),
            jax.ShapeDtypeStruct((n_tok, 2), jnp.int32),
        ],
    )(x2, W1, b1_eff, W2, b2r, ones)

    return (weights.reshape(batch, seq, _N_EXP),
            indices.reshape(batch, seq, 2))
